# Initial kernel scaffold; baseline (speedup 1.0000x reference)
import functools

import jax
import jax.numpy as jnp
from jax import lax
from jax.experimental import pallas as pl
from jax.experimental.pallas import tpu as pltpu
from jax.experimental.pallas import tpu_sc as plsc

N = 10000
D = 128
E = 320000

NC = 2
NS = 16
NW = NC * NS

NT = 10240
RPT = NT // NS
JUNK = N

K = 79
EPAD = NW * K * 128
EPT = K * 128
HG = EPT // 16

ROW_BLK = 1280
GRID = NT // ROW_BLK

_mesh = plsc.VectorSubcoreMesh(core_axis_name="c", subcore_axis_name="s")


def _hist_body(dst_hbm, out_hbm, dst_v, hist_v):
    c = lax.axis_index("c")
    s = lax.axis_index("s")
    wid = c * NS + s
    pltpu.sync_copy(dst_hbm.at[wid], dst_v)

    zeros = jnp.zeros((16,), jnp.float32)

    def zero_body(i, _):
        hist_v[pl.ds(i * 16, 16)] = zeros
        return 0

    lax.fori_loop(0, NT // 16, zero_body, 0)

    ones = jnp.ones((16,), jnp.float32)

    def body(j, _):
        idx = dst_v[j]
        plsc.addupdate_scatter(hist_v, [idx], ones)
        return 0

    lax.fori_loop(0, HG, body, 0)
    pltpu.sync_copy(hist_v, out_hbm.at[wid])


_hist_call = functools.partial(
    pl.kernel,
    out_type=jax.ShapeDtypeStruct((NW, NT), jnp.float32),
    mesh=_mesh,
    scratch_types=[
        pltpu.VMEM((HG, 16), jnp.int32),
        pltpu.VMEM((NT,), jnp.float32),
    ],
)(_hist_body)


def _prop_body(zs_hbm, src_hbm, dst_hbm, out_hbm, src_v, dst_v, rows_v,
               acc_sh, sem):
    c = lax.axis_index("c")
    s = lax.axis_index("s")
    wid = c * NS + s
    pltpu.sync_copy(src_hbm.at[wid], src_v)
    pltpu.sync_copy(dst_hbm.at[wid], dst_v)

    zeros = jnp.zeros((16,), jnp.float32)

    def zero_body(i, _):
        for kk in range(8):
            rows_v[i, pl.ds(kk * 16, 16)] = zeros
        return 0

    lax.fori_loop(0, 128, zero_body, 0)
    for kk in range(RPT // 128):
        pltpu.sync_copy(rows_v, acc_sh.at[pl.ds(s * RPT + kk * 128, 128)])
    plsc.subcore_barrier()

    def body(j, _):
        pltpu.async_copy(zs_hbm.at[src_v.at[j]], rows_v, sem).wait()
        pltpu.sync_copy(rows_v, acc_sh.at[dst_v.at[j]], add=True)
        return 0

    lax.fori_loop(0, K, body, 0)
    plsc.subcore_barrier()

    base = s * RPT
    pltpu.sync_copy(acc_sh.at[pl.ds(base, RPT)],
                    out_hbm.at[c].at[pl.ds(base, RPT)])


_prop_call = functools.partial(
    pl.kernel,
    out_type=jax.ShapeDtypeStruct((NC, NT, D), jnp.float32),
    mesh=_mesh,
    scratch_types=[
        pltpu.VMEM((K, 128), jnp.int32),
        pltpu.VMEM((K, 128), jnp.int32),
        pltpu.VMEM((128, D), jnp.float32),
        pltpu.VMEM_SHARED((NT, D), jnp.float32),
        pltpu.SemaphoreType.DMA,
    ],
)(_prop_body)


def _b1_body(x_ref, w_ref, ht_ref, zs_ref, dis_ref):
    deg = jnp.sum(ht_ref[...], axis=1, keepdims=True) + 1.0
    dis = lax.rsqrt(deg)
    xw = jnp.dot(x_ref[...], w_ref[...], preferred_element_type=jnp.float32)
    zs_ref[...] = xw * dis
    dis_ref[...] = dis


def _b1_call(x_pad, w1, hist_t):
    return pl.pallas_call(
        _b1_body,
        grid=(GRID,),
        in_specs=[
            pl.BlockSpec((ROW_BLK, D), lambda i: (i, 0)),
            pl.BlockSpec((D, D), lambda i: (0, 0)),
            pl.BlockSpec((ROW_BLK, NW), lambda i: (i, 0)),
        ],
        out_specs=[
            pl.BlockSpec((ROW_BLK, D), lambda i: (i, 0)),
            pl.BlockSpec((ROW_BLK, 1), lambda i: (i, 0)),
        ],
        out_shape=[
            jax.ShapeDtypeStruct((NT, D), jnp.float32),
            jax.ShapeDtypeStruct((NT, 1), jnp.float32),
        ],
    )(x_pad, w1, hist_t)


def _b2_body(s0_ref, s1_ref, zs1_ref, dis_ref, b1_ref, zs2_ref):
    pre = dis_ref[...] * (s0_ref[...] + s1_ref[...] + zs1_ref[...]) + b1_ref[...]
    zs2_ref[...] = dis_ref[...] * jnp.maximum(pre, 0.0)


def _b2_call(s0, s1, zs1, dis, b1_row):
    return pl.pallas_call(
        _b2_body,
        grid=(GRID,),
        in_specs=[
            pl.BlockSpec((ROW_BLK, D), lambda i: (i, 0)),
            pl.BlockSpec((ROW_BLK, D), lambda i: (i, 0)),
            pl.BlockSpec((ROW_BLK, D), lambda i: (i, 0)),
            pl.BlockSpec((ROW_BLK, 1), lambda i: (i, 0)),
            pl.BlockSpec((1, D), lambda i: (0, 0)),
        ],
        out_specs=pl.BlockSpec((ROW_BLK, D), lambda i: (i, 0)),
        out_shape=jax.ShapeDtypeStruct((NT, D), jnp.float32),
    )(s0, s1, zs1, dis, b1_row)


def _b3_body(t0_ref, t1_ref, zs2_ref, dis_ref, wmu_ref, bmu_ref, wlv_ref,
             blv_ref, mu_ref, lv_ref):
    agg = dis_ref[...] * (t0_ref[...] + t1_ref[...] + zs2_ref[...])
    mu_ref[...] = jnp.dot(agg, wmu_ref[...],
                          preferred_element_type=jnp.float32) + bmu_ref[...]
    lv_ref[...] = jnp.dot(agg, wlv_ref[...],
                          preferred_element_type=jnp.float32) + blv_ref[...]


def _b3_call(t0, t1, zs2, dis, wmu, bmu_row, wlv, blv_row):
    return pl.pallas_call(
        _b3_body,
        grid=(GRID,),
        in_specs=[
            pl.BlockSpec((ROW_BLK, D), lambda i: (i, 0)),
            pl.BlockSpec((ROW_BLK, D), lambda i: (i, 0)),
            pl.BlockSpec((ROW_BLK, D), lambda i: (i, 0)),
            pl.BlockSpec((ROW_BLK, 1), lambda i: (i, 0)),
            pl.BlockSpec((D, D), lambda i: (0, 0)),
            pl.BlockSpec((1, D), lambda i: (0, 0)),
            pl.BlockSpec((D, D), lambda i: (0, 0)),
            pl.BlockSpec((1, D), lambda i: (0, 0)),
        ],
        out_specs=[
            pl.BlockSpec((ROW_BLK, D), lambda i: (i, 0)),
            pl.BlockSpec((ROW_BLK, D), lambda i: (i, 0)),
        ],
        out_shape=[
            jax.ShapeDtypeStruct((NT, D), jnp.float32),
            jax.ShapeDtypeStruct((NT, D), jnp.float32),
        ],
    )(t0, t1, zs2, dis, wmu, bmu_row, wlv, blv_row)


def kernel(x_list, edge_index, W1, b1, W_mu, b_mu, W_lv, b_lv):
    x_pad = jnp.zeros((NT, D), jnp.float32).at[:N].set(x_list)
    pad = jnp.full((EPAD - E,), JUNK, jnp.int32)
    srcp_flat = jnp.concatenate([edge_index[0], pad])
    dstp_flat = jnp.concatenate([edge_index[1], pad])
    srcp = srcp_flat.reshape(NW, K, 128)
    dstp = dstp_flat.reshape(NW, K, 128)
    dsth = dstp_flat.reshape(NW, HG, 16)

    hist = _hist_call(dsth)
    zs1, dis = _b1_call(x_pad, W1, hist.T)
    s = _prop_call(zs1, srcp, dstp)
    zs2 = _b2_call(s[0], s[1], zs1, dis, b1.reshape(1, D))
    t = _prop_call(zs2, srcp, dstp)
    mu_pad, lv_pad = _b3_call(t[0], t[1], zs2, dis, W_mu,
                              b_mu.reshape(1, D), W_lv, b_lv.reshape(1, D))
    return (mu_pad[:N], lv_pad[:N])


# speedup vs baseline: 14.8542x; 14.8542x over previous
import functools

import jax
import jax.numpy as jnp
from jax import lax
from jax.experimental import pallas as pl
from jax.experimental.pallas import tpu as pltpu
from jax.experimental.pallas import tpu_sc as plsc

N = 10000
D = 128
E = 320000

NC = 2
NS = 16
NW = NC * NS

NT = 10240
RPT = NT // NS
JUNK = N

K = 79
EPAD = NW * K * 128
EPT = K * 128
HG = EPT // 16

ROW_BLK = 1280
GRID = NT // ROW_BLK

_mesh = plsc.VectorSubcoreMesh(core_axis_name="c", subcore_axis_name="s")


def _hist_body(dst_hbm, out_hbm, dst_v, hist_v):
    c = lax.axis_index("c")
    s = lax.axis_index("s")
    wid = c * NS + s
    pltpu.sync_copy(dst_hbm.at[wid], dst_v)

    zeros = jnp.zeros((16,), jnp.float32)

    def zero_body(i, _):
        hist_v[pl.ds(i * 16, 16)] = zeros
        return 0

    lax.fori_loop(0, NT // 16, zero_body, 0)

    ones = jnp.ones((16,), jnp.float32)

    def body(j, _):
        idx = dst_v[j]
        plsc.addupdate_scatter(hist_v, [idx], ones)
        return 0

    lax.fori_loop(0, HG, body, 0)
    pltpu.sync_copy(hist_v, out_hbm.at[wid])


_hist_call = functools.partial(
    pl.kernel,
    out_type=jax.ShapeDtypeStruct((NW, NT), jnp.float32),
    mesh=_mesh,
    scratch_types=[
        pltpu.VMEM((HG, 16), jnp.int32),
        pltpu.VMEM((NT,), jnp.float32),
    ],
    compiler_params=pltpu.CompilerParams(needs_layout_passes=False),
)(_hist_body)


def _prop_body(zs_hbm, src_hbm, dst_hbm, out_hbm, src_v, dst_v, rows_v,
               acc_sh, sem):
    c = lax.axis_index("c")
    s = lax.axis_index("s")
    wid = c * NS + s
    pltpu.sync_copy(src_hbm.at[wid], src_v)
    pltpu.sync_copy(dst_hbm.at[wid], dst_v)

    zeros = jnp.zeros((16,), jnp.float32)

    def zero_body(i, _):
        for kk in range(8):
            rows_v[i, pl.ds(kk * 16, 16)] = zeros
        return 0

    lax.fori_loop(0, 128, zero_body, 0)
    for kk in range(RPT // 128):
        pltpu.sync_copy(rows_v, acc_sh.at[pl.ds(s * RPT + kk * 128, 128)])
    plsc.subcore_barrier()

    def body(j, _):
        pltpu.async_copy(zs_hbm.at[src_v.at[j]], rows_v, sem).wait()
        pltpu.sync_copy(rows_v, acc_sh.at[dst_v.at[j]], add=True)
        return 0

    lax.fori_loop(0, K, body, 0)
    plsc.subcore_barrier()

    base = s * RPT
    pltpu.sync_copy(acc_sh.at[pl.ds(base, RPT)],
                    out_hbm.at[c].at[pl.ds(base, RPT)])


_prop_call = functools.partial(
    pl.kernel,
    out_type=jax.ShapeDtypeStruct((NC, NT, D), jnp.float32),
    mesh=_mesh,
    scratch_types=[
        pltpu.VMEM((K, 128), jnp.int32),
        pltpu.VMEM((K, 128), jnp.int32),
        pltpu.VMEM((128, D), jnp.float32),
        pltpu.VMEM_SHARED((NT, D), jnp.float32),
        pltpu.SemaphoreType.DMA,
    ],
)(_prop_body)


def _b1_body(x_ref, w_ref, ht_ref, zs_ref, dis_ref):
    deg = jnp.sum(ht_ref[...], axis=1, keepdims=True) + 1.0
    dis = lax.rsqrt(deg)
    xw = jnp.dot(x_ref[...], w_ref[...], preferred_element_type=jnp.float32)
    zs_ref[...] = xw * dis
    dis_ref[...] = dis


def _b1_call(x_pad, w1, hist_t):
    return pl.pallas_call(
        _b1_body,
        grid=(GRID,),
        in_specs=[
            pl.BlockSpec((ROW_BLK, D), lambda i: (i, 0)),
            pl.BlockSpec((D, D), lambda i: (0, 0)),
            pl.BlockSpec((ROW_BLK, NW), lambda i: (i, 0)),
        ],
        out_specs=[
            pl.BlockSpec((ROW_BLK, D), lambda i: (i, 0)),
            pl.BlockSpec((ROW_BLK, 1), lambda i: (i, 0)),
        ],
        out_shape=[
            jax.ShapeDtypeStruct((NT, D), jnp.float32),
            jax.ShapeDtypeStruct((NT, 1), jnp.float32),
        ],
    )(x_pad, w1, hist_t)


def _b2_body(s0_ref, s1_ref, zs1_ref, dis_ref, b1_ref, zs2_ref):
    pre = dis_ref[...] * (s0_ref[...] + s1_ref[...] + zs1_ref[...]) + b1_ref[...]
    zs2_ref[...] = dis_ref[...] * jnp.maximum(pre, 0.0)


def _b2_call(s0, s1, zs1, dis, b1_row):
    return pl.pallas_call(
        _b2_body,
        grid=(GRID,),
        in_specs=[
            pl.BlockSpec((ROW_BLK, D), lambda i: (i, 0)),
            pl.BlockSpec((ROW_BLK, D), lambda i: (i, 0)),
            pl.BlockSpec((ROW_BLK, D), lambda i: (i, 0)),
            pl.BlockSpec((ROW_BLK, 1), lambda i: (i, 0)),
            pl.BlockSpec((1, D), lambda i: (0, 0)),
        ],
        out_specs=pl.BlockSpec((ROW_BLK, D), lambda i: (i, 0)),
        out_shape=jax.ShapeDtypeStruct((NT, D), jnp.float32),
    )(s0, s1, zs1, dis, b1_row)


def _b3_body(t0_ref, t1_ref, zs2_ref, dis_ref, wmu_ref, bmu_ref, wlv_ref,
             blv_ref, mu_ref, lv_ref):
    agg = dis_ref[...] * (t0_ref[...] + t1_ref[...] + zs2_ref[...])
    mu_ref[...] = jnp.dot(agg, wmu_ref[...],
                          preferred_element_type=jnp.float32) + bmu_ref[...]
    lv_ref[...] = jnp.dot(agg, wlv_ref[...],
                          preferred_element_type=jnp.float32) + blv_ref[...]


def _b3_call(t0, t1, zs2, dis, wmu, bmu_row, wlv, blv_row):
    return pl.pallas_call(
        _b3_body,
        grid=(GRID,),
        in_specs=[
            pl.BlockSpec((ROW_BLK, D), lambda i: (i, 0)),
            pl.BlockSpec((ROW_BLK, D), lambda i: (i, 0)),
            pl.BlockSpec((ROW_BLK, D), lambda i: (i, 0)),
            pl.BlockSpec((ROW_BLK, 1), lambda i: (i, 0)),
            pl.BlockSpec((D, D), lambda i: (0, 0)),
            pl.BlockSpec((1, D), lambda i: (0, 0)),
            pl.BlockSpec((D, D), lambda i: (0, 0)),
            pl.BlockSpec((1, D), lambda i: (0, 0)),
        ],
        out_specs=[
            pl.BlockSpec((ROW_BLK, D), lambda i: (i, 0)),
            pl.BlockSpec((ROW_BLK, D), lambda i: (i, 0)),
        ],
        out_shape=[
            jax.ShapeDtypeStruct((NT, D), jnp.float32),
            jax.ShapeDtypeStruct((NT, D), jnp.float32),
        ],
    )(t0, t1, zs2, dis, wmu, bmu_row, wlv, blv_row)


def kernel(x_list, edge_index, W1, b1, W_mu, b_mu, W_lv, b_lv):
    x_pad = jnp.zeros((NT, D), jnp.float32).at[:N].set(x_list)
    pad = jnp.full((EPAD - E,), JUNK, jnp.int32)
    srcp_flat = jnp.concatenate([edge_index[0], pad])
    dstp_flat = jnp.concatenate([edge_index[1], pad])
    srcp = srcp_flat.reshape(NW, K, 128)
    dstp = dstp_flat.reshape(NW, K, 128)
    dsth = dstp_flat.reshape(NW, HG, 16)

    hist = _hist_call(dsth)
    zs1, dis = _b1_call(x_pad, W1, hist.T)
    s = _prop_call(zs1, srcp, dstp)
    zs2 = _b2_call(s[0], s[1], zs1, dis, b1.reshape(1, D))
    t = _prop_call(zs2, srcp, dstp)
    mu_pad, lv_pad = _b3_call(t[0], t[1], zs2, dis, W_mu,
                              b_mu.reshape(1, D), W_lv, b_lv.reshape(1, D))
    return (mu_pad[:N], lv_pad[:N])


# speedup vs baseline: 27.8621x; 1.8757x over previous
import functools

import jax
import jax.numpy as jnp
from jax import lax
from jax.experimental import pallas as pl
from jax.experimental.pallas import tpu as pltpu
from jax.experimental.pallas import tpu_sc as plsc

N = 10000
D = 128
E = 320000

NC = 2
NS = 16
NW = NC * NS

CS = D // NC
NT = 10240
RPT = NT // NS

CH = 80
K = 250
EPT = K * CH
HG = E // NW // 16

ROW_BLK = 1280
GRID = NT // ROW_BLK

_mesh = plsc.VectorSubcoreMesh(core_axis_name="c", subcore_axis_name="s")


def _hist_body(dst_hbm, out_hbm, dst_v, hist_v):
    c = lax.axis_index("c")
    s = lax.axis_index("s")
    wid = c * NS + s
    pltpu.sync_copy(dst_hbm.at[wid], dst_v)

    zeros = jnp.zeros((16,), jnp.float32)

    def zero_body(i, _):
        hist_v[pl.ds(i * 16, 16)] = zeros
        return 0

    lax.fori_loop(0, NT // 16, zero_body, 0)

    ones = jnp.ones((16,), jnp.float32)

    def body(j, _):
        idx = dst_v[j]
        plsc.addupdate_scatter(hist_v, [idx], ones)
        return 0

    lax.fori_loop(0, HG, body, 0)
    pltpu.sync_copy(hist_v, out_hbm.at[wid])


_hist_call = functools.partial(
    pl.kernel,
    out_type=jax.ShapeDtypeStruct((NW, NT), jnp.float32),
    mesh=_mesh,
    scratch_types=[
        pltpu.VMEM((HG, 16), jnp.int32),
        pltpu.VMEM((NT,), jnp.float32),
    ],
    compiler_params=pltpu.CompilerParams(needs_layout_passes=False),
)(_hist_body)


def _prop_body(zs_hbm, src_hbm, dst_hbm, out_hbm, src_v, dst_v, rows0, rows1,
               acc_sh, sem0, sem1):
    c = lax.axis_index("c")
    s = lax.axis_index("s")
    pltpu.sync_copy(src_hbm.at[c, s], src_v)
    pltpu.sync_copy(dst_hbm.at[s], dst_v)

    zeros = jnp.zeros((16,), jnp.float32)

    def zero_body(i, _):
        for kk in range(CS // 16):
            rows0[i, pl.ds(kk * 16, 16)] = zeros
        return 0

    lax.fori_loop(0, CH, zero_body, 0)
    for kk in range(RPT // CH):
        pltpu.sync_copy(rows0, acc_sh.at[pl.ds(s * RPT + kk * CH, CH)])
    plsc.subcore_barrier()

    pltpu.async_copy(zs_hbm.at[src_v.at[0]], rows0, sem0)

    def pipe(i, _):
        j = 2 * i
        pltpu.async_copy(zs_hbm.at[src_v.at[j + 1]], rows1, sem1)
        pltpu.make_async_copy(zs_hbm.at[src_v.at[j]], rows0, sem0).wait()
        pltpu.sync_copy(rows0, acc_sh.at[dst_v.at[j]], add=True)

        @pl.when(j + 2 < K)
        def _():
            pltpu.async_copy(zs_hbm.at[src_v.at[j + 2]], rows0, sem0)

        pltpu.make_async_copy(zs_hbm.at[src_v.at[j + 1]], rows1, sem1).wait()
        pltpu.sync_copy(rows1, acc_sh.at[dst_v.at[j + 1]], add=True)
        return 0

    lax.fori_loop(0, K // 2, pipe, 0)
    plsc.subcore_barrier()

    base = s * RPT
    pltpu.sync_copy(acc_sh.at[pl.ds(base, RPT)],
                    out_hbm.at[c].at[pl.ds(base, RPT)])


_prop_call = functools.partial(
    pl.kernel,
    out_type=jax.ShapeDtypeStruct((NC, NT, CS), jnp.float32),
    mesh=_mesh,
    scratch_types=[
        pltpu.VMEM((K, CH), jnp.int32),
        pltpu.VMEM((K, CH), jnp.int32),
        pltpu.VMEM((CH, CS), jnp.float32),
        pltpu.VMEM((CH, CS), jnp.float32),
        pltpu.VMEM_SHARED((NT, CS), jnp.float32),
        pltpu.SemaphoreType.DMA,
        pltpu.SemaphoreType.DMA,
    ],
    compiler_params=pltpu.CompilerParams(use_tc_tiling_on_sc=False),
)(_prop_body)


def _b1_body(x_ref, w_ref, ht_ref, zs_ref, dis_ref):
    deg = jnp.sum(ht_ref[...], axis=1, keepdims=True) + 1.0
    dis = lax.rsqrt(deg)
    xw = jnp.dot(x_ref[...], w_ref[...], preferred_element_type=jnp.float32)
    zsw = xw * dis
    zs_ref[0] = zsw[:, :CS]
    zs_ref[1] = zsw[:, CS:]
    dis_ref[...] = dis


def _b1_call(x_pad, w1, hist_t):
    return pl.pallas_call(
        _b1_body,
        grid=(GRID,),
        in_specs=[
            pl.BlockSpec((ROW_BLK, D), lambda i: (i, 0)),
            pl.BlockSpec((D, D), lambda i: (0, 0)),
            pl.BlockSpec((ROW_BLK, NW), lambda i: (i, 0)),
        ],
        out_specs=[
            pl.BlockSpec((NC, ROW_BLK, CS), lambda i: (0, i, 0)),
            pl.BlockSpec((ROW_BLK, 1), lambda i: (i, 0)),
        ],
        out_shape=[
            jax.ShapeDtypeStruct((NC, NT, CS), jnp.float32),
            jax.ShapeDtypeStruct((NT, 1), jnp.float32),
        ],
    )(x_pad, w1, hist_t)


def _b2_body(s_ref, zs1_ref, dis_ref, b1_ref, zs2_ref):
    pre = dis_ref[...] * (s_ref[...] + zs1_ref[...]) + b1_ref[...]
    zs2_ref[...] = dis_ref[...] * jnp.maximum(pre, 0.0)


def _b2_call(s, zs1, dis, b1_split):
    return pl.pallas_call(
        _b2_body,
        grid=(GRID,),
        in_specs=[
            pl.BlockSpec((NC, ROW_BLK, CS), lambda i: (0, i, 0)),
            pl.BlockSpec((NC, ROW_BLK, CS), lambda i: (0, i, 0)),
            pl.BlockSpec((ROW_BLK, 1), lambda i: (i, 0)),
            pl.BlockSpec((NC, 1, CS), lambda i: (0, 0, 0)),
        ],
        out_specs=pl.BlockSpec((NC, ROW_BLK, CS), lambda i: (0, i, 0)),
        out_shape=jax.ShapeDtypeStruct((NC, NT, CS), jnp.float32),
    )(s, zs1, dis, b1_split)


def _b3_body(t_ref, zs2_ref, dis_ref, wmu_ref, bmu_ref, wlv_ref, blv_ref,
             mu_ref, lv_ref):
    agg3 = dis_ref[...] * (t_ref[...] + zs2_ref[...])
    agg = jnp.concatenate([agg3[0], agg3[1]], axis=1)
    mu_ref[...] = jnp.dot(agg, wmu_ref[...],
                          preferred_element_type=jnp.float32) + bmu_ref[...]
    lv_ref[...] = jnp.dot(agg, wlv_ref[...],
                          preferred_element_type=jnp.float32) + blv_ref[...]


def _b3_call(t, zs2, dis, wmu, bmu_row, wlv, blv_row):
    return pl.pallas_call(
        _b3_body,
        grid=(GRID,),
        in_specs=[
            pl.BlockSpec((NC, ROW_BLK, CS), lambda i: (0, i, 0)),
            pl.BlockSpec((NC, ROW_BLK, CS), lambda i: (0, i, 0)),
            pl.BlockSpec((ROW_BLK, 1), lambda i: (i, 0)),
            pl.BlockSpec((D, D), lambda i: (0, 0)),
            pl.BlockSpec((1, D), lambda i: (0, 0)),
            pl.BlockSpec((D, D), lambda i: (0, 0)),
            pl.BlockSpec((1, D), lambda i: (0, 0)),
        ],
        out_specs=[
            pl.BlockSpec((ROW_BLK, D), lambda i: (i, 0)),
            pl.BlockSpec((ROW_BLK, D), lambda i: (i, 0)),
        ],
        out_shape=[
            jax.ShapeDtypeStruct((NT, D), jnp.float32),
            jax.ShapeDtypeStruct((NT, D), jnp.float32),
        ],
    )(t, zs2, dis, wmu, bmu_row, wlv, blv_row)


def kernel(x_list, edge_index, W1, b1, W_mu, b_mu, W_lv, b_lv):
    x_pad = jnp.zeros((NT, D), jnp.float32).at[:N].set(x_list)
    srcp = edge_index[0].reshape(NS, K, CH)
    src2 = jnp.stack([srcp, srcp + NT])
    dstp = edge_index[1].reshape(NS, K, CH)
    dsth = edge_index[1].reshape(NW, HG, 16)

    hist = _hist_call(dsth)
    zs1, dis = _b1_call(x_pad, W1, hist.T)
    s = _prop_call(zs1.reshape(NC * NT, CS), src2, dstp)
    zs2 = _b2_call(s, zs1, dis, b1.reshape(NC, 1, CS))
    t = _prop_call(zs2.reshape(NC * NT, CS), src2, dstp)
    mu_pad, lv_pad = _b3_call(t, zs2, dis, W_mu, b_mu.reshape(1, D),
                              W_lv, b_lv.reshape(1, D))
    return (mu_pad[:N], lv_pad[:N])


# speedup vs baseline: 39.3631x; 1.4128x over previous
import functools

import jax
import jax.numpy as jnp
from jax import lax
from jax.experimental import pallas as pl
from jax.experimental.pallas import tpu as pltpu
from jax.experimental.pallas import tpu_sc as plsc

N = 10000
D = 128
E = 320000

NC = 2
NS = 16
NW = NC * NS

CS = D // NC
RPT = N // NS

CH = 80
K = 250
NB = 5
HG = E // NW // 16

ROW_BLK = 2000
GRID = N // ROW_BLK

_mesh = plsc.VectorSubcoreMesh(core_axis_name="c", subcore_axis_name="s")


def _hist_body(dst_hbm, out_hbm, dst_v, hist_v):
    c = lax.axis_index("c")
    s = lax.axis_index("s")
    wid = c * NS + s
    pltpu.sync_copy(dst_hbm.at[wid], dst_v)

    zeros = jnp.zeros((16,), jnp.float32)

    def zero_body(i, _):
        hist_v[pl.ds(i * 16, 16)] = zeros
        return 0

    lax.fori_loop(0, N // 16, zero_body, 0)

    ones = jnp.ones((16,), jnp.float32)

    def body(j, _):
        idx = dst_v[j]
        plsc.addupdate_scatter(hist_v, [idx], ones)
        return 0

    lax.fori_loop(0, HG, body, 0)
    pltpu.sync_copy(hist_v, out_hbm.at[wid])


_hist_call = functools.partial(
    pl.kernel,
    out_type=jax.ShapeDtypeStruct((NW, N), jnp.float32),
    mesh=_mesh,
    scratch_types=[
        pltpu.VMEM((HG, 16), jnp.int32),
        pltpu.VMEM((N,), jnp.float32),
    ],
    compiler_params=pltpu.CompilerParams(needs_layout_passes=False),
)(_hist_body)


def _prop_body(zs_hbm, src_hbm, dst_hbm, out_hbm, src_v, dst_v,
               r0, r1, r2, r3, r4, acc_sh,
               g0, g1, g2, g3, g4, t0, t1, t2, t3, t4):
    rows = [r0, r1, r2, r3, r4]
    gsem = [g0, g1, g2, g3, g4]
    ssem = [t0, t1, t2, t3, t4]
    c = lax.axis_index("c")
    s = lax.axis_index("s")
    pltpu.sync_copy(src_hbm.at[c, s], src_v)
    pltpu.sync_copy(dst_hbm.at[s], dst_v)

    zeros = jnp.zeros((16,), jnp.float32)

    def zero_body(i, _):
        for kk in range(CS // 16):
            r0[i, pl.ds(kk * 16, 16)] = zeros
        return 0

    lax.fori_loop(0, CH, zero_body, 0)
    base = s * RPT
    for kk in range(RPT // CH):
        pltpu.sync_copy(r0, acc_sh.at[pl.ds(base + kk * CH, CH)])
    rem = RPT % CH
    if rem:
        pltpu.sync_copy(r0.at[pl.ds(0, rem)],
                        acc_sh.at[pl.ds(base + (RPT // CH) * CH, rem)])
    plsc.subcore_barrier()

    for j in range(3):
        pltpu.async_copy(zs_hbm.at[src_v.at[j]], rows[j], gsem[j])

    def slot(j, b):
        bg = (b + 3) % NB

        @pl.when(j + 3 < K)
        def _():
            @pl.when(j >= 2)
            def _():
                pltpu.make_async_copy(
                    rows[bg], acc_sh.at[dst_v.at[j - 2]], ssem[bg],
                ).wait()

            pltpu.async_copy(zs_hbm.at[src_v.at[j + 3]], rows[bg], gsem[bg])

        pltpu.make_async_copy(zs_hbm.at[src_v.at[j]], rows[b], gsem[b]).wait()
        pltpu.async_copy(rows[b], acc_sh.at[dst_v.at[j]], ssem[b], add=True)

    def pipe(i, _):
        for b in range(NB):
            slot(i * NB + b, b)
        return 0

    lax.fori_loop(0, K // NB, pipe, 0)
    for b in range(NB):
        jj = K - NB + b
        pltpu.make_async_copy(rows[b], acc_sh.at[dst_v.at[jj]], ssem[b]).wait()
    plsc.subcore_barrier()

    pltpu.sync_copy(acc_sh.at[pl.ds(base, RPT)],
                    out_hbm.at[c].at[pl.ds(base, RPT)])


_prop_call = functools.partial(
    pl.kernel,
    out_type=jax.ShapeDtypeStruct((NC, N, CS), jnp.float32),
    mesh=_mesh,
    scratch_types=(
        [pltpu.VMEM((K, CH), jnp.int32), pltpu.VMEM((K, CH), jnp.int32)]
        + [pltpu.VMEM((CH, CS), jnp.float32)] * NB
        + [pltpu.VMEM_SHARED((N, CS), jnp.float32)]
        + [pltpu.SemaphoreType.DMA] * (2 * NB)
    ),
    compiler_params=pltpu.CompilerParams(use_tc_tiling_on_sc=False),
)(_prop_body)


def _b1_body(x_ref, w_ref, ht_ref, zs_ref, dis_ref):
    deg = jnp.sum(ht_ref[...], axis=1, keepdims=True) + 1.0
    dis = lax.rsqrt(deg)
    xw = jnp.dot(x_ref[...], w_ref[...], preferred_element_type=jnp.float32)
    zsw = xw * dis
    zs_ref[0] = zsw[:, :CS]
    zs_ref[1] = zsw[:, CS:]
    dis_ref[...] = dis


def _b1_call(x, w1, hist_t):
    return pl.pallas_call(
        _b1_body,
        grid=(GRID,),
        in_specs=[
            pl.BlockSpec((ROW_BLK, D), lambda i: (i, 0)),
            pl.BlockSpec((D, D), lambda i: (0, 0)),
            pl.BlockSpec((ROW_BLK, NW), lambda i: (i, 0)),
        ],
        out_specs=[
            pl.BlockSpec((NC, ROW_BLK, CS), lambda i: (0, i, 0)),
            pl.BlockSpec((ROW_BLK, 1), lambda i: (i, 0)),
        ],
        out_shape=[
            jax.ShapeDtypeStruct((NC, N, CS), jnp.float32),
            jax.ShapeDtypeStruct((N, 1), jnp.float32),
        ],
    )(x, w1, hist_t)


def _b2_body(s_ref, zs1_ref, dis_ref, b1_ref, zs2_ref):
    pre = dis_ref[...] * (s_ref[...] + zs1_ref[...]) + b1_ref[...]
    zs2_ref[...] = dis_ref[...] * jnp.maximum(pre, 0.0)


def _b2_call(s, zs1, dis, b1_split):
    return pl.pallas_call(
        _b2_body,
        grid=(GRID,),
        in_specs=[
            pl.BlockSpec((NC, ROW_BLK, CS), lambda i: (0, i, 0)),
            pl.BlockSpec((NC, ROW_BLK, CS), lambda i: (0, i, 0)),
            pl.BlockSpec((ROW_BLK, 1), lambda i: (i, 0)),
            pl.BlockSpec((NC, 1, CS), lambda i: (0, 0, 0)),
        ],
        out_specs=pl.BlockSpec((NC, ROW_BLK, CS), lambda i: (0, i, 0)),
        out_shape=jax.ShapeDtypeStruct((NC, N, CS), jnp.float32),
    )(s, zs1, dis, b1_split)


def _b3_body(t_ref, zs2_ref, dis_ref, wmu_ref, bmu_ref, wlv_ref, blv_ref,
             mu_ref, lv_ref):
    agg3 = dis_ref[...] * (t_ref[...] + zs2_ref[...])
    agg = jnp.concatenate([agg3[0], agg3[1]], axis=1)
    mu_ref[...] = jnp.dot(agg, wmu_ref[...],
                          preferred_element_type=jnp.float32) + bmu_ref[...]
    lv_ref[...] = jnp.dot(agg, wlv_ref[...],
                          preferred_element_type=jnp.float32) + blv_ref[...]


def _b3_call(t, zs2, dis, wmu, bmu_row, wlv, blv_row):
    return pl.pallas_call(
        _b3_body,
        grid=(GRID,),
        in_specs=[
            pl.BlockSpec((NC, ROW_BLK, CS), lambda i: (0, i, 0)),
            pl.BlockSpec((NC, ROW_BLK, CS), lambda i: (0, i, 0)),
            pl.BlockSpec((ROW_BLK, 1), lambda i: (i, 0)),
            pl.BlockSpec((D, D), lambda i: (0, 0)),
            pl.BlockSpec((1, D), lambda i: (0, 0)),
            pl.BlockSpec((D, D), lambda i: (0, 0)),
            pl.BlockSpec((1, D), lambda i: (0, 0)),
        ],
        out_specs=[
            pl.BlockSpec((ROW_BLK, D), lambda i: (i, 0)),
            pl.BlockSpec((ROW_BLK, D), lambda i: (i, 0)),
        ],
        out_shape=[
            jax.ShapeDtypeStruct((N, D), jnp.float32),
            jax.ShapeDtypeStruct((N, D), jnp.float32),
        ],
    )(t, zs2, dis, wmu, bmu_row, wlv, blv_row)


def kernel(x_list, edge_index, W1, b1, W_mu, b_mu, W_lv, b_lv):
    srcp = edge_index[0].reshape(NS, K, CH)
    src2 = jnp.stack([srcp, srcp + N])
    dstp = edge_index[1].reshape(NS, K, CH)
    dsth = edge_index[1].reshape(NW, HG, 16)

    hist = _hist_call(dsth)
    zs1, dis = _b1_call(x_list, W1, hist.T)
    s = _prop_call(zs1.reshape(NC * N, CS), src2, dstp)
    zs2 = _b2_call(s, zs1, dis, b1.reshape(NC, 1, CS))
    t = _prop_call(zs2.reshape(NC * N, CS), src2, dstp)
    mu, lv = _b3_call(t, zs2, dis, W_mu, b_mu.reshape(1, D),
                      W_lv, b_lv.reshape(1, D))
    return (mu, lv)


# speedup vs baseline: 40.6167x; 1.0318x over previous
import functools

import jax
import jax.numpy as jnp
from jax import lax
from jax.experimental import pallas as pl
from jax.experimental.pallas import tpu as pltpu
from jax.experimental.pallas import tpu_sc as plsc

N = 10000
D = 128
E = 320000

NC = 2
NS = 16
NW = NC * NS

CS = D // NC
RPT = N // NS

CH = 80
K = 250
NB = 5
HG = E // NW // 16

ROW_BLK = 2000
GRID = N // ROW_BLK

_mesh = plsc.VectorSubcoreMesh(core_axis_name="c", subcore_axis_name="s")


def _hist_body(dst_hbm, out_hbm, dst_v, hist_v):
    c = lax.axis_index("c")
    s = lax.axis_index("s")
    wid = c * NS + s
    pltpu.sync_copy(dst_hbm.at[wid], dst_v)

    zeros = jnp.zeros((16,), jnp.float32)

    def zero_body(i, _):
        hist_v[pl.ds(i * 16, 16)] = zeros
        return 0

    lax.fori_loop(0, N // 16, zero_body, 0)

    ones = jnp.ones((16,), jnp.float32)

    def body(j, _):
        jr = j // (CH // 16)
        jc = (j % (CH // 16)) * 16
        idx = dst_v[jr, pl.ds(jc, 16)]
        plsc.addupdate_scatter(hist_v, [idx], ones)
        return 0

    lax.fori_loop(0, HG, body, 0)
    pltpu.sync_copy(hist_v, out_hbm.at[wid])


_hist_call = functools.partial(
    pl.kernel,
    out_type=jax.ShapeDtypeStruct((NW, N), jnp.float32),
    mesh=_mesh,
    scratch_types=[
        pltpu.VMEM((K // 2, CH), jnp.int32),
        pltpu.VMEM((N,), jnp.float32),
    ],
    compiler_params=pltpu.CompilerParams(needs_layout_passes=False),
)(_hist_body)


def _prop_body(zs_hbm, src_hbm, dst_hbm, out_hbm, src_v, dst_v,
               r0, r1, r2, r3, r4, acc_sh,
               g0, g1, g2, g3, g4, t0, t1, t2, t3, t4):
    rows = [r0, r1, r2, r3, r4]
    gsem = [g0, g1, g2, g3, g4]
    ssem = [t0, t1, t2, t3, t4]
    c = lax.axis_index("c")
    s = lax.axis_index("s")
    pltpu.sync_copy(src_hbm.at[c, s], src_v)
    pltpu.sync_copy(dst_hbm.at[s], dst_v)

    zeros = jnp.zeros((16,), jnp.float32)

    def zero_body(i, _):
        for kk in range(CS // 16):
            r0[i, pl.ds(kk * 16, 16)] = zeros
        return 0

    lax.fori_loop(0, CH, zero_body, 0)
    base = s * RPT
    for kk in range(RPT // CH):
        pltpu.sync_copy(r0, acc_sh.at[pl.ds(base + kk * CH, CH)])
    rem = RPT % CH
    if rem:
        pltpu.sync_copy(r0.at[pl.ds(0, rem)],
                        acc_sh.at[pl.ds(base + (RPT // CH) * CH, rem)])
    plsc.subcore_barrier()

    for j in range(3):
        pltpu.async_copy(zs_hbm.at[src_v.at[j]], rows[j], gsem[j])

    def slot(j, b):
        bg = (b + 3) % NB

        @pl.when(j + 3 < K)
        def _():
            @pl.when(j >= 2)
            def _():
                pltpu.make_async_copy(
                    rows[bg], acc_sh.at[dst_v.at[j - 2]], ssem[bg],
                ).wait()

            pltpu.async_copy(zs_hbm.at[src_v.at[j + 3]], rows[bg], gsem[bg])

        pltpu.make_async_copy(zs_hbm.at[src_v.at[j]], rows[b], gsem[b]).wait()
        pltpu.async_copy(rows[b], acc_sh.at[dst_v.at[j]], ssem[b], add=True)

    def pipe(i, _):
        for b in range(NB):
            slot(i * NB + b, b)
        return 0

    lax.fori_loop(0, K // NB, pipe, 0)
    for b in range(NB):
        jj = K - NB + b
        pltpu.make_async_copy(rows[b], acc_sh.at[dst_v.at[jj]], ssem[b]).wait()
    plsc.subcore_barrier()

    pltpu.sync_copy(acc_sh.at[pl.ds(base, RPT)],
                    out_hbm.at[c].at[pl.ds(base, RPT)])


_prop_call = functools.partial(
    pl.kernel,
    out_type=jax.ShapeDtypeStruct((NC, N, CS), jnp.float32),
    mesh=_mesh,
    scratch_types=(
        [pltpu.VMEM((K, CH), jnp.int32), pltpu.VMEM((K, CH), jnp.int32)]
        + [pltpu.VMEM((CH, CS), jnp.float32)] * NB
        + [pltpu.VMEM_SHARED((N, CS), jnp.float32)]
        + [pltpu.SemaphoreType.DMA] * (2 * NB)
    ),
    compiler_params=pltpu.CompilerParams(use_tc_tiling_on_sc=False),
)(_prop_body)


def _b1_body(x_ref, w_ref, ht_ref, zs_ref, dis_ref):
    deg = jnp.sum(ht_ref[...], axis=1, keepdims=True) + 1.0
    dis = lax.rsqrt(deg)
    xw = jnp.dot(x_ref[...], w_ref[...], preferred_element_type=jnp.float32)
    zsw = xw * dis
    zs_ref[0] = zsw[:, :CS]
    zs_ref[1] = zsw[:, CS:]
    dis_ref[...] = dis


def _b1_call(x, w1, hist_t):
    return pl.pallas_call(
        _b1_body,
        grid=(GRID,),
        in_specs=[
            pl.BlockSpec((ROW_BLK, D), lambda i: (i, 0)),
            pl.BlockSpec((D, D), lambda i: (0, 0)),
            pl.BlockSpec((ROW_BLK, NW), lambda i: (i, 0)),
        ],
        out_specs=[
            pl.BlockSpec((NC, ROW_BLK, CS), lambda i: (0, i, 0)),
            pl.BlockSpec((ROW_BLK, 1), lambda i: (i, 0)),
        ],
        out_shape=[
            jax.ShapeDtypeStruct((NC, N, CS), jnp.float32),
            jax.ShapeDtypeStruct((N, 1), jnp.float32),
        ],
    )(x, w1, hist_t)


def _b2_body(s_ref, zs1_ref, dis_ref, b1_ref, zs2_ref):
    pre = dis_ref[...] * (s_ref[...] + zs1_ref[...]) + b1_ref[...]
    zs2_ref[...] = dis_ref[...] * jnp.maximum(pre, 0.0)


def _b2_call(s, zs1, dis, b1_split):
    return pl.pallas_call(
        _b2_body,
        grid=(GRID,),
        in_specs=[
            pl.BlockSpec((NC, ROW_BLK, CS), lambda i: (0, i, 0)),
            pl.BlockSpec((NC, ROW_BLK, CS), lambda i: (0, i, 0)),
            pl.BlockSpec((ROW_BLK, 1), lambda i: (i, 0)),
            pl.BlockSpec((NC, 1, CS), lambda i: (0, 0, 0)),
        ],
        out_specs=pl.BlockSpec((NC, ROW_BLK, CS), lambda i: (0, i, 0)),
        out_shape=jax.ShapeDtypeStruct((NC, N, CS), jnp.float32),
    )(s, zs1, dis, b1_split)


def _b3_body(t_ref, zs2_ref, dis_ref, wmu_ref, bmu_ref, wlv_ref, blv_ref,
             mu_ref, lv_ref):
    agg3 = dis_ref[...] * (t_ref[...] + zs2_ref[...])
    agg = jnp.concatenate([agg3[0], agg3[1]], axis=1)
    mu_ref[...] = jnp.dot(agg, wmu_ref[...],
                          preferred_element_type=jnp.float32) + bmu_ref[...]
    lv_ref[...] = jnp.dot(agg, wlv_ref[...],
                          preferred_element_type=jnp.float32) + blv_ref[...]


def _b3_call(t, zs2, dis, wmu, bmu_row, wlv, blv_row):
    return pl.pallas_call(
        _b3_body,
        grid=(GRID,),
        in_specs=[
            pl.BlockSpec((NC, ROW_BLK, CS), lambda i: (0, i, 0)),
            pl.BlockSpec((NC, ROW_BLK, CS), lambda i: (0, i, 0)),
            pl.BlockSpec((ROW_BLK, 1), lambda i: (i, 0)),
            pl.BlockSpec((D, D), lambda i: (0, 0)),
            pl.BlockSpec((1, D), lambda i: (0, 0)),
            pl.BlockSpec((D, D), lambda i: (0, 0)),
            pl.BlockSpec((1, D), lambda i: (0, 0)),
        ],
        out_specs=[
            pl.BlockSpec((ROW_BLK, D), lambda i: (i, 0)),
            pl.BlockSpec((ROW_BLK, D), lambda i: (i, 0)),
        ],
        out_shape=[
            jax.ShapeDtypeStruct((N, D), jnp.float32),
            jax.ShapeDtypeStruct((N, D), jnp.float32),
        ],
    )(t, zs2, dis, wmu, bmu_row, wlv, blv_row)


def kernel(x_list, edge_index, W1, b1, W_mu, b_mu, W_lv, b_lv):
    srcp = edge_index[0].reshape(NS, K, CH)
    src2 = jnp.stack([srcp, srcp + N])
    dstp = edge_index[1].reshape(NS, K, CH)
    dsth = edge_index[1].reshape(NW, K // 2, CH)

    hist = _hist_call(dsth)
    zs1, dis = _b1_call(x_list, W1, hist.T)
    s = _prop_call(zs1.reshape(NC * N, CS), src2, dstp)
    zs2 = _b2_call(s, zs1, dis, b1.reshape(NC, 1, CS))
    t = _prop_call(zs2.reshape(NC * N, CS), src2, dstp)
    mu, lv = _b3_call(t, zs2, dis, W_mu, b_mu.reshape(1, D),
                      W_lv, b_lv.reshape(1, D))
    return (mu, lv)


# speedup vs baseline: 41.0131x; 1.0098x over previous
import functools

import jax
import jax.numpy as jnp
from jax import lax
from jax.experimental import pallas as pl
from jax.experimental.pallas import tpu as pltpu
from jax.experimental.pallas import tpu_sc as plsc

N = 10000
D = 128
E = 320000

NC = 2
NS = 16
NW = NC * NS

CS = D // NC
RPT = N // NS

CH = 125
K = 160
NB = 5
HR = 125
HC = 80
HG = HR * HC // 16

ROW_BLK = 2000
GRID = N // ROW_BLK

_mesh = plsc.VectorSubcoreMesh(core_axis_name="c", subcore_axis_name="s")


def _hist_body(dst_hbm, out_hbm, dst_v, hist_v):
    c = lax.axis_index("c")
    s = lax.axis_index("s")
    wid = c * NS + s
    pltpu.sync_copy(dst_hbm.at[wid], dst_v)

    zeros = jnp.zeros((16,), jnp.float32)

    def zero_body(i, _):
        hist_v[pl.ds(i * 16, 16)] = zeros
        return 0

    lax.fori_loop(0, N // 16, zero_body, 0)

    ones = jnp.ones((16,), jnp.float32)

    def body(j, _):
        jr = j // (HC // 16)
        jc = (j % (HC // 16)) * 16
        idx = dst_v[jr, pl.ds(jc, 16)]
        plsc.addupdate_scatter(hist_v, [idx], ones)
        return 0

    lax.fori_loop(0, HG, body, 0)
    pltpu.sync_copy(hist_v, out_hbm.at[wid])


_hist_call = functools.partial(
    pl.kernel,
    out_type=jax.ShapeDtypeStruct((NW, N), jnp.float32),
    mesh=_mesh,
    scratch_types=[
        pltpu.VMEM((HR, HC), jnp.int32),
        pltpu.VMEM((N,), jnp.float32),
    ],
    compiler_params=pltpu.CompilerParams(needs_layout_passes=False),
)(_hist_body)


def _prop_body(zs_hbm, src_hbm, dst_hbm, out_hbm, src_v, dst_v,
               r0, r1, r2, r3, r4, acc_sh,
               g0, g1, g2, g3, g4, t0, t1, t2, t3, t4):
    rows = [r0, r1, r2, r3, r4]
    gsem = [g0, g1, g2, g3, g4]
    ssem = [t0, t1, t2, t3, t4]
    c = lax.axis_index("c")
    s = lax.axis_index("s")
    pltpu.sync_copy(src_hbm.at[c, s], src_v)
    pltpu.sync_copy(dst_hbm.at[s], dst_v)

    zeros = jnp.zeros((16,), jnp.float32)

    def zero_body(i, _):
        for kk in range(CS // 16):
            r0[i, pl.ds(kk * 16, 16)] = zeros
        return 0

    lax.fori_loop(0, CH, zero_body, 0)
    base = s * RPT
    for kk in range(RPT // CH):
        pltpu.sync_copy(r0, acc_sh.at[pl.ds(base + kk * CH, CH)])
    rem = RPT % CH
    if rem:
        pltpu.sync_copy(r0.at[pl.ds(0, rem)],
                        acc_sh.at[pl.ds(base + (RPT // CH) * CH, rem)])
    plsc.subcore_barrier()

    for j in range(3):
        pltpu.async_copy(zs_hbm.at[src_v.at[j]], rows[j], gsem[j])

    def slot(j, b):
        bg = (b + 3) % NB

        @pl.when(j + 3 < K)
        def _():
            @pl.when(j >= 2)
            def _():
                pltpu.make_async_copy(
                    rows[bg], acc_sh.at[dst_v.at[j - 2]], ssem[bg],
                ).wait()

            pltpu.async_copy(zs_hbm.at[src_v.at[j + 3]], rows[bg], gsem[bg])

        pltpu.make_async_copy(zs_hbm.at[src_v.at[j]], rows[b], gsem[b]).wait()
        pltpu.async_copy(rows[b], acc_sh.at[dst_v.at[j]], ssem[b], add=True)

    def pipe(i, _):
        for b in range(NB):
            slot(i * NB + b, b)
        return 0

    lax.fori_loop(0, K // NB, pipe, 0)
    for b in range(NB):
        jj = K - NB + b
        pltpu.make_async_copy(rows[b], acc_sh.at[dst_v.at[jj]], ssem[b]).wait()
    plsc.subcore_barrier()

    pltpu.sync_copy(acc_sh.at[pl.ds(base, RPT)],
                    out_hbm.at[c].at[pl.ds(base, RPT)])


_prop_call = functools.partial(
    pl.kernel,
    out_type=jax.ShapeDtypeStruct((NC, N, CS), jnp.float32),
    mesh=_mesh,
    scratch_types=(
        [pltpu.VMEM((K, CH), jnp.int32), pltpu.VMEM((K, CH), jnp.int32)]
        + [pltpu.VMEM((CH, CS), jnp.float32)] * NB
        + [pltpu.VMEM_SHARED((N, CS), jnp.float32)]
        + [pltpu.SemaphoreType.DMA] * (2 * NB)
    ),
    compiler_params=pltpu.CompilerParams(use_tc_tiling_on_sc=False),
)(_prop_body)


def _b1_body(x_ref, w_ref, ht_ref, zs_ref, dis_ref):
    deg = jnp.sum(ht_ref[...], axis=1, keepdims=True) + 1.0
    dis = lax.rsqrt(deg)
    xw = jnp.dot(x_ref[...], w_ref[...], preferred_element_type=jnp.float32)
    zsw = xw * dis
    zs_ref[0] = zsw[:, :CS]
    zs_ref[1] = zsw[:, CS:]
    dis_ref[...] = dis


def _b1_call(x, w1, hist_t):
    return pl.pallas_call(
        _b1_body,
        grid=(GRID,),
        in_specs=[
            pl.BlockSpec((ROW_BLK, D), lambda i: (i, 0)),
            pl.BlockSpec((D, D), lambda i: (0, 0)),
            pl.BlockSpec((ROW_BLK, NW), lambda i: (i, 0)),
        ],
        out_specs=[
            pl.BlockSpec((NC, ROW_BLK, CS), lambda i: (0, i, 0)),
            pl.BlockSpec((ROW_BLK, 1), lambda i: (i, 0)),
        ],
        out_shape=[
            jax.ShapeDtypeStruct((NC, N, CS), jnp.float32),
            jax.ShapeDtypeStruct((N, 1), jnp.float32),
        ],
    )(x, w1, hist_t)


def _b2_body(s_ref, zs1_ref, dis_ref, b1_ref, zs2_ref):
    pre = dis_ref[...] * (s_ref[...] + zs1_ref[...]) + b1_ref[...]
    zs2_ref[...] = dis_ref[...] * jnp.maximum(pre, 0.0)


def _b2_call(s, zs1, dis, b1_split):
    return pl.pallas_call(
        _b2_body,
        grid=(GRID,),
        in_specs=[
            pl.BlockSpec((NC, ROW_BLK, CS), lambda i: (0, i, 0)),
            pl.BlockSpec((NC, ROW_BLK, CS), lambda i: (0, i, 0)),
            pl.BlockSpec((ROW_BLK, 1), lambda i: (i, 0)),
            pl.BlockSpec((NC, 1, CS), lambda i: (0, 0, 0)),
        ],
        out_specs=pl.BlockSpec((NC, ROW_BLK, CS), lambda i: (0, i, 0)),
        out_shape=jax.ShapeDtypeStruct((NC, N, CS), jnp.float32),
    )(s, zs1, dis, b1_split)


def _b3_body(t_ref, zs2_ref, dis_ref, wmu_ref, bmu_ref, wlv_ref, blv_ref,
             mu_ref, lv_ref):
    agg3 = dis_ref[...] * (t_ref[...] + zs2_ref[...])
    agg = jnp.concatenate([agg3[0], agg3[1]], axis=1)
    mu_ref[...] = jnp.dot(agg, wmu_ref[...],
                          preferred_element_type=jnp.float32) + bmu_ref[...]
    lv_ref[...] = jnp.dot(agg, wlv_ref[...],
                          preferred_element_type=jnp.float32) + blv_ref[...]


def _b3_call(t, zs2, dis, wmu, bmu_row, wlv, blv_row):
    return pl.pallas_call(
        _b3_body,
        grid=(GRID,),
        in_specs=[
            pl.BlockSpec((NC, ROW_BLK, CS), lambda i: (0, i, 0)),
            pl.BlockSpec((NC, ROW_BLK, CS), lambda i: (0, i, 0)),
            pl.BlockSpec((ROW_BLK, 1), lambda i: (i, 0)),
            pl.BlockSpec((D, D), lambda i: (0, 0)),
            pl.BlockSpec((1, D), lambda i: (0, 0)),
            pl.BlockSpec((D, D), lambda i: (0, 0)),
            pl.BlockSpec((1, D), lambda i: (0, 0)),
        ],
        out_specs=[
            pl.BlockSpec((ROW_BLK, D), lambda i: (i, 0)),
            pl.BlockSpec((ROW_BLK, D), lambda i: (i, 0)),
        ],
        out_shape=[
            jax.ShapeDtypeStruct((N, D), jnp.float32),
            jax.ShapeDtypeStruct((N, D), jnp.float32),
        ],
    )(t, zs2, dis, wmu, bmu_row, wlv, blv_row)


def kernel(x_list, edge_index, W1, b1, W_mu, b_mu, W_lv, b_lv):
    srcp = edge_index[0].reshape(NS, K, CH)
    src2 = jnp.stack([srcp, srcp + N])
    dstp = edge_index[1].reshape(NS, K, CH)
    dsth = edge_index[1].reshape(NW, HR, HC)

    hist = _hist_call(dsth)
    zs1, dis = _b1_call(x_list, W1, hist.T)
    s = _prop_call(zs1.reshape(NC * N, CS), src2, dstp)
    zs2 = _b2_call(s, zs1, dis, b1.reshape(NC, 1, CS))
    t = _prop_call(zs2.reshape(NC * N, CS), src2, dstp)
    mu, lv = _b3_call(t, zs2, dis, W_mu, b_mu.reshape(1, D),
                      W_lv, b_lv.reshape(1, D))
    return (mu, lv)
